# Initial kernel scaffold; baseline (speedup 1.0000x reference)
import jax
import jax.numpy as jnp
from jax.experimental import pallas as pl


def kernel(input_ids, task_ids, segment_ids, word_table, task_table, segment_table):
    raise NotImplementedError("write your pallas kernel here")


# speedup vs baseline: 1.2900x; 1.2900x over previous
import functools
import math

import jax
import jax.numpy as jnp
from jax import lax
from jax.experimental import pallas as pl
from jax.experimental.pallas import tpu as pltpu
from jax.experimental.pallas import tpu_sc as plsc

D = 512
N_TOK = 8192
SCALE = 1.0 / math.sqrt(D)

_info = plsc.get_sparse_core_info()
_NC, _NS, _L = _info.num_cores, _info.num_subcores, _info.num_lanes
_NW = _NC * _NS
_TPW = N_TOK // _NW
_CH = 64
_NCHUNK = _TPW // _CH


def _emb_body(ids_hbm, tid_hbm, gid_hbm, word_hbm, task_hbm, seg_hbm,
              out_hbm, combt_hbm,
              idx_v, tid_v, gid_v, cidx_v, tt_v, st_v, comb_v,
              rows_v, crows_v, sem):
    cid = lax.axis_index("c")
    sid = lax.axis_index("s")
    base = (sid * _NC + cid) * _TPW

    @pl.when(sid == 0)
    def _build():
        pltpu.sync_copy(task_hbm, tt_v)
        pltpu.sync_copy(seg_hbm, st_v)

        def build9(j, carry):
            sl = pl.ds(j * _L, _L)
            for t in range(3):
                for g in range(3):
                    comb_v[t * 3 + g, sl] = tt_v[t, sl] + st_v[g, sl] * SCALE
            return carry

        lax.fori_loop(0, D // _L, build9, 0)
        pltpu.sync_copy(comb_v, combt_hbm.at[pl.ds(cid * 16, 16)])

    plsc.subcore_barrier()

    for c in range(_NCHUNK):
        off = base + c * _CH
        pltpu.sync_copy(ids_hbm.at[pl.ds(off, _CH)], idx_v)
        pltpu.sync_copy(tid_hbm.at[pl.ds(off, _CH)], tid_v)
        pltpu.sync_copy(gid_hbm.at[pl.ds(off, _CH)], gid_v)

        for j in range(_CH // _L):
            sl = pl.ds(j * _L, _L)
            cidx_v[sl] = tid_v[sl] * 3 + gid_v[sl] + cid * 16

        cp_w = pltpu.async_copy(word_hbm.at[idx_v], rows_v, sem)
        cp_c = pltpu.async_copy(combt_hbm.at[cidx_v], crows_v, sem)
        cp_w.wait()
        cp_c.wait()

        def tok(i, carry):
            for j in range(D // _L):
                sl = pl.ds(j * _L, _L)
                rows_v[i, sl] = rows_v[i, sl] + crows_v[i, sl]
            return carry

        lax.fori_loop(0, _CH, tok, 0)
        pltpu.sync_copy(rows_v, out_hbm.at[pl.ds(off, _CH)])


_emb_kernel = functools.partial(
    pl.kernel,
    out_type=(
        jax.ShapeDtypeStruct((N_TOK, D), jnp.float32),
        jax.ShapeDtypeStruct((2 * 16, D), jnp.float32),
    ),
    mesh=plsc.VectorSubcoreMesh(core_axis_name="c", subcore_axis_name="s"),
    scratch_types=[
        pltpu.VMEM((_CH,), jnp.int32),
        pltpu.VMEM((_CH,), jnp.int32),
        pltpu.VMEM((_CH,), jnp.int32),
        pltpu.VMEM((_CH,), jnp.int32),
        pltpu.VMEM((3, D), jnp.float32),
        pltpu.VMEM((3, D), jnp.float32),
        pltpu.VMEM((16, D), jnp.float32),
        pltpu.VMEM((_CH, D), jnp.float32),
        pltpu.VMEM((_CH, D), jnp.float32),
        pltpu.SemaphoreType.DMA,
    ],
)(_emb_body)


@jax.jit
def kernel(input_ids, task_ids, segment_ids, word_table, task_table,
           segment_table):
    shape = input_ids.shape
    ids = input_ids.reshape(-1).astype(jnp.int32)
    tid = task_ids.reshape(-1).astype(jnp.int32)
    gid = segment_ids.reshape(-1).astype(jnp.int32)
    out, _ = _emb_kernel(ids, tid, gid, word_table, task_table, segment_table)
    return out.reshape(shape + (D,))


# speedup vs baseline: 1.3255x; 1.0275x over previous
import functools
import math

import jax
import jax.numpy as jnp
from jax import lax
from jax.experimental import pallas as pl
from jax.experimental.pallas import tpu as pltpu
from jax.experimental.pallas import tpu_sc as plsc

D = 512
N_TOK = 8192
SCALE = 1.0 / math.sqrt(D)

_info = plsc.get_sparse_core_info()
_NC, _NS, _L = _info.num_cores, _info.num_subcores, _info.num_lanes
_NW = _NC * _NS
_TPW = N_TOK // _NW
_CH = 32
_NCHUNK = _TPW // _CH


def _emb_body(ids_hbm, tid_hbm, gid_hbm, word_hbm, task_hbm, seg_hbm,
              out_hbm, combt_hbm,
              idx_v, cidx_v, ident_v, tt_v, st_v, comb_v,
              rows_v0, rows_v1, crows_v0, crows_v1,
              gsem0, gsem1, osem0, osem1):
    cid = lax.axis_index("c")
    sid = lax.axis_index("s")
    base = (sid * _NC + cid) * _TPW
    rows = (rows_v0, rows_v1)
    crows = (crows_v0, crows_v1)
    gsem = (gsem0, gsem1)
    osem = (osem0, osem1)

    @pl.when(sid == 0)
    def _build():
        pltpu.sync_copy(task_hbm, tt_v)
        pltpu.sync_copy(seg_hbm, st_v)

        def build9(j, carry):
            sl = pl.ds(j * _L, _L)
            for t in range(3):
                for g in range(3):
                    comb_v[t * 3 + g, sl] = tt_v[t, sl] + st_v[g, sl] * SCALE
            return carry

        lax.fori_loop(0, D // _L, build9, 0)
        pltpu.sync_copy(comb_v, combt_hbm.at[pl.ds(cid * 16, 16)])

    pltpu.sync_copy(ids_hbm.at[pl.ds(base, _TPW)], idx_v)
    pltpu.sync_copy(tid_hbm.at[pl.ds(base, _TPW)], cidx_v)
    pltpu.sync_copy(gid_hbm.at[pl.ds(base, _TPW)], ident_v)
    for j in range(_TPW // _L):
        sl = pl.ds(j * _L, _L)
        cidx_v[sl] = cidx_v[sl] * 3 + ident_v[sl] + cid * 16

    plsc.subcore_barrier()

    cpw = [None, None]
    cpc = [None, None]
    cpo = [None, None]

    def start(c):
        b = c % 2
        if cpo[b] is not None:
            cpo[b].wait()
        cpw[b] = pltpu.async_copy(
            word_hbm.at[idx_v.at[pl.ds(c * _CH, _CH)]], rows[b], gsem[b])
        cpc[b] = pltpu.async_copy(
            combt_hbm.at[cidx_v.at[pl.ds(c * _CH, _CH)]], crows[b], gsem[b])

    start(0)
    for c in range(_NCHUNK):
        b = c % 2
        if c + 1 < _NCHUNK:
            start(c + 1)
        cpw[b].wait()
        cpc[b].wait()

        def tok(i, carry):
            for j in range(D // _L):
                sl = pl.ds(j * _L, _L)
                plsc.addupdate(rows[b].at[i, sl], crows[b][i, sl])
            return carry

        lax.fori_loop(0, _CH, tok, 0)
        cpo[b] = pltpu.async_copy(
            rows[b], out_hbm.at[pl.ds(base + c * _CH, _CH)], osem[b])
    cpo[0].wait()
    cpo[1].wait()


_emb_kernel = functools.partial(
    pl.kernel,
    out_type=(
        jax.ShapeDtypeStruct((N_TOK, D), jnp.float32),
        jax.ShapeDtypeStruct((2 * 16, D), jnp.float32),
    ),
    mesh=plsc.VectorSubcoreMesh(core_axis_name="c", subcore_axis_name="s"),
    scratch_types=[
        pltpu.VMEM((_TPW,), jnp.int32),
        pltpu.VMEM((_TPW,), jnp.int32),
        pltpu.VMEM((_TPW,), jnp.int32),
        pltpu.VMEM((3, D), jnp.float32),
        pltpu.VMEM((3, D), jnp.float32),
        pltpu.VMEM((16, D), jnp.float32),
        pltpu.VMEM((_CH, D), jnp.float32),
        pltpu.VMEM((_CH, D), jnp.float32),
        pltpu.VMEM((_CH, D), jnp.float32),
        pltpu.VMEM((_CH, D), jnp.float32),
        pltpu.SemaphoreType.DMA,
        pltpu.SemaphoreType.DMA,
        pltpu.SemaphoreType.DMA,
        pltpu.SemaphoreType.DMA,
    ],
)(_emb_body)


@jax.jit
def kernel(input_ids, task_ids, segment_ids, word_table, task_table,
           segment_table):
    shape = input_ids.shape
    ids = input_ids.reshape(-1).astype(jnp.int32)
    tid = task_ids.reshape(-1).astype(jnp.int32)
    gid = segment_ids.reshape(-1).astype(jnp.int32)
    out, _ = _emb_kernel(ids, tid, gid, word_table, task_table, segment_table)
    return out.reshape(shape + (D,))


# speedup vs baseline: 1.6155x; 1.2188x over previous
import functools
import math

import jax
import jax.numpy as jnp
from jax import lax
from jax.experimental import pallas as pl
from jax.experimental.pallas import tpu as pltpu
from jax.experimental.pallas import tpu_sc as plsc

D = 512
N_TOK = 8192
SCALE = 1.0 / math.sqrt(D)

_info = plsc.get_sparse_core_info()
_NC, _NS, _L = _info.num_cores, _info.num_subcores, _info.num_lanes
_NW = _NC * _NS
_TPW = N_TOK // _NW
_CH = 64
_NCHUNK = _TPW // _CH


def _emb_body(ids_hbm, tid_hbm, gid_hbm, word_hbm, task_hbm, seg_hbm,
              out_hbm,
              idx_v, cidx_v, tmp_v, tt_v, st_v, comb_v,
              rows_v0, rows_v1, gsem0, gsem1, osem0, osem1):
    cid = lax.axis_index("c")
    sid = lax.axis_index("s")
    base = (sid * _NC + cid) * _TPW
    rows = (rows_v0, rows_v1)
    gsem = (gsem0, gsem1)
    osem = (osem0, osem1)

    pltpu.sync_copy(task_hbm, tt_v)
    pltpu.sync_copy(seg_hbm, st_v)

    def build9(j, carry):
        sl = pl.ds(j * _L, _L)
        for t in range(3):
            for g in range(3):
                comb_v[t * 3 + g, sl] = tt_v[t, sl] + st_v[g, sl] * SCALE
        return carry

    lax.fori_loop(0, D // _L, build9, 0)

    pltpu.sync_copy(ids_hbm.at[pl.ds(base, _TPW)], idx_v)
    pltpu.sync_copy(tid_hbm.at[pl.ds(base, _TPW)], cidx_v.at[pl.ds(0, _TPW)])
    pltpu.sync_copy(gid_hbm.at[pl.ds(base, _TPW)], tmp_v)
    for j in range(_TPW // _L):
        sl = pl.ds(j * _L, _L)
        cidx_v[sl] = cidx_v[sl] * 3 + tmp_v[sl]

    lane = lax.iota(jnp.int32, _L)
    cpw = [None, None]
    cpo = [None, None]

    def start(c):
        b = c % 2
        if cpo[b] is not None:
            cpo[b].wait()
        cpw[b] = pltpu.async_copy(
            word_hbm.at[idx_v.at[pl.ds(c * _CH, _CH)]], rows[b], gsem[b])

    start(0)
    for c in range(_NCHUNK):
        b = c % 2
        if c + 1 < _NCHUNK:
            start(c + 1)
        cpw[b].wait()

        def tok(i, carry, b=b, c=c):
            cc = cidx_v[pl.ds(c * _CH + i, _L)][0]
            for j in range(D // _L):
                sl = pl.ds(j * _L, _L)
                plsc.addupdate(rows[b].at[i, sl], comb_v[cc, sl])
            return carry

        lax.fori_loop(0, _CH, tok, 0)
        cpo[b] = pltpu.async_copy(
            rows[b], out_hbm.at[pl.ds(base + c * _CH, _CH)], osem[b])
    cpo[0].wait()
    cpo[1].wait()


_emb_kernel = functools.partial(
    pl.kernel,
    out_type=jax.ShapeDtypeStruct((N_TOK, D), jnp.float32),
    mesh=plsc.VectorSubcoreMesh(core_axis_name="c", subcore_axis_name="s"),
    scratch_types=[
        pltpu.VMEM((_TPW,), jnp.int32),
        pltpu.VMEM((_TPW + _L,), jnp.int32),
        pltpu.VMEM((_TPW,), jnp.int32),
        pltpu.VMEM((3, D), jnp.float32),
        pltpu.VMEM((3, D), jnp.float32),
        pltpu.VMEM((9, D), jnp.float32),
        pltpu.VMEM((_CH, D), jnp.float32),
        pltpu.VMEM((_CH, D), jnp.float32),
        pltpu.SemaphoreType.DMA,
        pltpu.SemaphoreType.DMA,
        pltpu.SemaphoreType.DMA,
        pltpu.SemaphoreType.DMA,
    ],
)(_emb_body)


@jax.jit
def kernel(input_ids, task_ids, segment_ids, word_table, task_table,
           segment_table):
    shape = input_ids.shape
    ids = input_ids.reshape(-1).astype(jnp.int32)
    tid = task_ids.reshape(-1).astype(jnp.int32)
    gid = segment_ids.reshape(-1).astype(jnp.int32)
    out = _emb_kernel(ids, tid, gid, word_table, task_table, segment_table)
    return out.reshape(shape + (D,))


# speedup vs baseline: 2.3296x; 1.4420x over previous
import functools
import math

import jax
import jax.numpy as jnp
from jax import lax
from jax.experimental import pallas as pl
from jax.experimental.pallas import tpu as pltpu
from jax.experimental.pallas import tpu_sc as plsc

D = 512
N_TOK = 8192
SCALE = 1.0 / math.sqrt(D)

_info = plsc.get_sparse_core_info()
_NC, _NS, _L = _info.num_cores, _info.num_subcores, _info.num_lanes
_NW = _NC * _NS
_TPW = N_TOK // _NW
_CH = 64
_NCHUNK = _TPW // _CH


def _emb_body(ids_hbm, tid_hbm, gid_hbm, word_hbm, task_hbm, seg_hbm,
              out_hbm,
              idx_v, cidx_v, tmp_v, tt_v, st_v, comb_v,
              rows_v0, rows_v1, gsem0, gsem1, osem0, osem1):
    cid = lax.axis_index("c")
    sid = lax.axis_index("s")
    base = (sid * _NC + cid) * _TPW
    rows = (rows_v0, rows_v1)
    gsem = (gsem0, gsem1)
    osem = (osem0, osem1)

    pltpu.sync_copy(task_hbm, tt_v)
    pltpu.sync_copy(seg_hbm, st_v)

    def build9(j, carry):
        sl = pl.ds(j * _L, _L)
        for t in range(3):
            for g in range(3):
                comb_v[t * 3 + g, sl] = tt_v[t, sl] + st_v[g, sl] * SCALE
        return carry

    lax.fori_loop(0, D // _L, build9, 0)

    pltpu.sync_copy(ids_hbm.at[pl.ds(base, _TPW)], idx_v)
    pltpu.sync_copy(tid_hbm.at[pl.ds(base, _TPW)], cidx_v.at[pl.ds(0, _TPW)])
    pltpu.sync_copy(gid_hbm.at[pl.ds(base, _TPW)], tmp_v)
    for j in range(_TPW // _L):
        sl = pl.ds(j * _L, _L)
        cidx_v[sl] = cidx_v[sl] * 3 + tmp_v[sl]

    lane = lax.iota(jnp.int32, _L)
    cpw = [None, None]
    cpo = [None, None]

    def start(c):
        b = c % 2
        if cpo[b] is not None:
            cpo[b].wait()
        cpw[b] = pltpu.async_copy(
            word_hbm.at[idx_v.at[pl.ds(c * _CH, _CH)]], rows[b], gsem[b])

    start(0)
    for c in range(_NCHUNK):
        b = c % 2
        if c + 1 < _NCHUNK:
            start(c + 1)
        cpw[b].wait()

        @plsc.parallel_loop(0, _CH, unroll=2)
        def tok(i, b=b, c=c):
            cc = cidx_v[pl.ds(c * _CH + i, _L)][0]
            for j in range(D // _L):
                sl = pl.ds(j * _L, _L)
                plsc.addupdate(rows[b].at[i, sl], comb_v[cc, sl])
        cpo[b] = pltpu.async_copy(
            rows[b], out_hbm.at[pl.ds(base + c * _CH, _CH)], osem[b])
    cpo[0].wait()
    cpo[1].wait()


_emb_kernel = functools.partial(
    pl.kernel,
    out_type=jax.ShapeDtypeStruct((N_TOK, D), jnp.float32),
    mesh=plsc.VectorSubcoreMesh(core_axis_name="c", subcore_axis_name="s"),
    scratch_types=[
        pltpu.VMEM((_TPW,), jnp.int32),
        pltpu.VMEM((_TPW + _L,), jnp.int32),
        pltpu.VMEM((_TPW,), jnp.int32),
        pltpu.VMEM((3, D), jnp.float32),
        pltpu.VMEM((3, D), jnp.float32),
        pltpu.VMEM((9, D), jnp.float32),
        pltpu.VMEM((_CH, D), jnp.float32),
        pltpu.VMEM((_CH, D), jnp.float32),
        pltpu.SemaphoreType.DMA,
        pltpu.SemaphoreType.DMA,
        pltpu.SemaphoreType.DMA,
        pltpu.SemaphoreType.DMA,
    ],
)(_emb_body)


@jax.jit
def kernel(input_ids, task_ids, segment_ids, word_table, task_table,
           segment_table):
    shape = input_ids.shape
    ids = input_ids.reshape(-1).astype(jnp.int32)
    tid = task_ids.reshape(-1).astype(jnp.int32)
    gid = segment_ids.reshape(-1).astype(jnp.int32)
    out = _emb_kernel(ids, tid, gid, word_table, task_table, segment_table)
    return out.reshape(shape + (D,))
